# Initial kernel scaffold; baseline (speedup 1.0000x reference)
import jax
import jax.numpy as jnp
from jax.experimental import pallas as pl


def kernel(user_emb, item_emb, graph_indices, graph_values):
    raise NotImplementedError("write your pallas kernel here")


# speedup vs baseline: 2.5624x; 2.5624x over previous
import functools

import jax
import jax.numpy as jnp
from jax import lax
from jax.experimental import pallas as pl
from jax.experimental.pallas import tpu as pltpu
from jax.experimental.pallas import tpu_sc as plsc

N_TOTAL = 50000
D = 64
DH = 32
NNZ = 800000
N_LAYERS = 3
NS = 16
EDGES_PER_TILE = NNZ // NS
EB = 80
N_EBLK = EDGES_PER_TILE // EB
ROWS_PER_TILE = N_TOTAL // NS
RB = 125
N_RBLK = ROWS_PER_TILE // RB

_mesh = plsc.VectorSubcoreMesh(core_axis_name="c", subcore_axis_name="s")

_f32 = jnp.float32
_out = jax.ShapeDtypeStruct((N_TOTAL, DH), _f32)


@functools.partial(
    pl.kernel,
    out_type=(_out, _out, _out, _out, _out, _out),
    mesh=_mesh,
    compiler_params=pltpu.CompilerParams(use_tc_tiling_on_sc=False),
    scratch_types=[
        pltpu.VMEM_SHARED((N_TOTAL, DH), _f32),
        pltpu.VMEM((EB,), jnp.int32),
        pltpu.VMEM((EB,), jnp.int32),
        pltpu.VMEM((EB,), _f32),
        pltpu.VMEM((EB, DH), _f32),
        pltpu.VMEM((RB, DH), _f32),
        pltpu.VMEM((RB, DH), _f32),
        pltpu.VMEM((RB, DH), _f32),
        pltpu.VMEM((RB, DH), _f32),
    ],
)
def _lightgcn_sc(col_hbm, row_hbm, val_hbm, x_lo, x_hi,
                 fin_lo, fin_hi, l1_lo, l1_hi, l2_lo, l2_hi,
                 acc, col_v, row_v, val_v, msg_v, za_v, zb_v, zc_v, zd_v):
  cid = lax.axis_index("c")
  tid = lax.axis_index("s")
  ebase = tid * EDGES_PER_TILE
  rbase = tid * ROWS_PER_TILE
  zeros16 = jnp.zeros((16,), _f32)

  def fill_zeros(buf):
    def body(r, _):
      buf[r, pl.ds(0, 16)] = zeros16
      buf[r, pl.ds(16, 16)] = zeros16
      return 0
    lax.fori_loop(0, RB, body, 0)

  def zero_acc_slice():
    def body(j, _):
      pltpu.sync_copy(za_v, acc.at[pl.ds(rbase + j * RB, RB)])
      return 0
    lax.fori_loop(0, N_RBLK, body, 0)

  def edge_pass(src_hbm):
    def block(i, _):
      base = ebase + i * EB
      pltpu.sync_copy(col_hbm.at[pl.ds(base, EB)], col_v)
      pltpu.sync_copy(row_hbm.at[pl.ds(base, EB)], row_v)
      pltpu.sync_copy(val_hbm.at[pl.ds(base, EB)], val_v)
      pltpu.sync_copy(src_hbm.at[col_v], msg_v)
      def scale(g, _):
        vv = val_v[pl.ds(g * 16, 16)]
        e0 = g * 16
        for j in range(16):
          v = vv[j]
          msg_v[e0 + j, pl.ds(0, 16)] = msg_v[e0 + j, pl.ds(0, 16)] * v
          msg_v[e0 + j, pl.ds(16, 16)] = msg_v[e0 + j, pl.ds(16, 16)] * v
        return 0
      lax.fori_loop(0, EB // 16, scale, 0)
      pltpu.sync_copy(msg_v, acc.at[row_v], add=True)
      return 0
    lax.fori_loop(0, N_EBLK, block, 0)

  def writeback(dst_hbm):
    pltpu.sync_copy(acc.at[pl.ds(rbase, ROWS_PER_TILE)],
                    dst_hbm.at[pl.ds(rbase, ROWS_PER_TILE)])

  def combine(x_hbm, l1_hbm, l2_hbm, fin_hbm):
    def chunk(j, _):
      rs = rbase + j * RB
      pltpu.sync_copy(x_hbm.at[pl.ds(rs, RB)], za_v)
      pltpu.sync_copy(l1_hbm.at[pl.ds(rs, RB)], zb_v)
      pltpu.sync_copy(l2_hbm.at[pl.ds(rs, RB)], zc_v)
      pltpu.sync_copy(acc.at[pl.ds(rs, RB)], zd_v)
      def body(r, _):
        for h in (0, 16):
          s = (za_v[r, pl.ds(h, 16)] + zb_v[r, pl.ds(h, 16)]
               + zc_v[r, pl.ds(h, 16)] + zd_v[r, pl.ds(h, 16)])
          za_v[r, pl.ds(h, 16)] = s * 0.25
        return 0
      lax.fori_loop(0, RB, body, 0)
      pltpu.sync_copy(za_v, fin_hbm.at[pl.ds(rs, RB)])
      return 0
    lax.fori_loop(0, N_RBLK, chunk, 0)

  def propagate(x_hbm, l1_hbm, l2_hbm, fin_hbm):
    fill_zeros(za_v)
    zero_acc_slice()
    plsc.subcore_barrier()
    edge_pass(x_hbm)
    plsc.subcore_barrier()
    writeback(l1_hbm)
    fill_zeros(za_v)
    zero_acc_slice()
    plsc.subcore_barrier()
    edge_pass(l1_hbm)
    plsc.subcore_barrier()
    writeback(l2_hbm)
    fill_zeros(za_v)
    zero_acc_slice()
    plsc.subcore_barrier()
    edge_pass(l2_hbm)
    plsc.subcore_barrier()
    combine(x_hbm, l1_hbm, l2_hbm, fin_hbm)

  @pl.when(cid == 0)
  def _():
    propagate(x_lo, l1_lo, l2_lo, fin_lo)

  @pl.when(cid == 1)
  def _():
    propagate(x_hi, l1_hi, l2_hi, fin_hi)


def kernel(user_emb, item_emb, graph_indices, graph_values):
  all_emb = jnp.concatenate([user_emb, item_emb], axis=0)
  x_lo = all_emb[:, :DH]
  x_hi = all_emb[:, DH:]
  row = graph_indices[0]
  col = graph_indices[1]
  fin_lo, fin_hi, _, _, _, _ = _lightgcn_sc(col, row, graph_values,
                                            x_lo, x_hi)
  light_out = jnp.concatenate([fin_lo, fin_hi], axis=1)
  n_users = user_emb.shape[0]
  return light_out[:n_users], light_out[n_users:]


# speedup vs baseline: 9.1920x; 3.5873x over previous
import functools

import jax
import jax.numpy as jnp
from jax import lax
from jax.experimental import pallas as pl
from jax.experimental.pallas import tpu as pltpu
from jax.experimental.pallas import tpu_sc as plsc

N_TOTAL = 50000
D = 64
DH = 32
NNZ = 800000
NS = 16
EB = 80
BLKS_PER_TILE = NNZ // NS // EB
NBUF = 5
CH = 25
N_CHUNK = BLKS_PER_TILE // CH
OUTER = CH // NBUF
ROWS_PER_TILE = N_TOTAL // NS
RB = 25
N_RBLK = ROWS_PER_TILE // RB
ZR = 125
N_ZBLK = ROWS_PER_TILE // ZR

_mesh = plsc.VectorSubcoreMesh(core_axis_name="c", subcore_axis_name="s")

_f32 = jnp.float32
_out = jax.ShapeDtypeStruct((N_TOTAL, DH), _f32)


@functools.partial(
    pl.kernel,
    out_type=(_out, _out, _out, _out, _out, _out),
    mesh=_mesh,
    compiler_params=pltpu.CompilerParams(use_tc_tiling_on_sc=False),
    scratch_types=[
        pltpu.VMEM_SHARED((N_TOTAL, DH), _f32),
        pltpu.VMEM((CH, EB), jnp.int32),
        pltpu.VMEM((CH, EB), jnp.int32),
        pltpu.VMEM((CH, EB), _f32),
        [pltpu.VMEM((EB, DH), _f32)] * NBUF,
        [pltpu.SemaphoreType.DMA] * NBUF,
        pltpu.VMEM((ZR, DH), _f32),
        pltpu.VMEM((RB, DH), _f32),
        pltpu.VMEM((RB, DH), _f32),
        pltpu.VMEM((RB, DH), _f32),
        pltpu.VMEM((RB, DH), _f32),
    ],
)
def _lightgcn_sc(col_hbm, row_hbm, val_hbm, x_lo, x_hi,
                 fin_lo, fin_hi, l1_lo, l1_hi, l2_lo, l2_hi,
                 acc, col_v, row_v, val_v, ga, gs, zz_v,
                 za_v, zb_v, zc_v, zd_v):
  cid = lax.axis_index("c")
  tid = lax.axis_index("s")
  rbase = tid * ROWS_PER_TILE
  zeros16 = jnp.zeros((16,), _f32)

  def fill_zeros(buf):
    def body(r, _):
      buf[r, pl.ds(0, 16)] = zeros16
      buf[r, pl.ds(16, 16)] = zeros16
      return 0
    lax.fori_loop(0, ZR, body, 0)

  def zero_acc_slice():
    def body(j, _):
      pltpu.sync_copy(zz_v, acc.at[pl.ds(rbase + j * ZR, ZR)])
      return 0
    lax.fori_loop(0, N_ZBLK, body, 0)

  def scale(buf, vrow):
    def grp(g, _):
      vv = val_v[vrow, pl.ds(g * 16, 16)]
      e0 = g * 16
      for j in range(16):
        v = vv[j]
        buf[e0 + j, pl.ds(0, 16)] = buf[e0 + j, pl.ds(0, 16)] * v
        buf[e0 + j, pl.ds(16, 16)] = buf[e0 + j, pl.ds(16, 16)] * v
      return 0
    lax.fori_loop(0, EB // 16, grp, 0)

  def edge_pass(src_hbm):
    def chunk(ci, _):
      crow = tid * BLKS_PER_TILE + ci * CH
      pltpu.sync_copy(col_hbm.at[pl.ds(crow, CH)], col_v)
      pltpu.sync_copy(row_hbm.at[pl.ds(crow, CH)], row_v)
      pltpu.sync_copy(val_hbm.at[pl.ds(crow, CH)], val_v)
      for k in range(NBUF):
        pltpu.async_copy(src_hbm.at[col_v.at[k]], ga[k], gs[k])
      def outer(oi, _):
        for k in range(NBUF):
          j = oi * NBUF + k
          pltpu.make_async_copy(src_hbm.at[col_v.at[j]], ga[k], gs[k]).wait()
          scale(ga[k], j)
          pltpu.sync_copy(ga[k], acc.at[row_v.at[j]], add=True)
          @pl.when(oi < OUTER - 1)
          def _():
            pltpu.async_copy(src_hbm.at[col_v.at[j + NBUF]], ga[k], gs[k])
        return 0
      lax.fori_loop(0, OUTER, outer, 0)
      return 0
    lax.fori_loop(0, N_CHUNK, chunk, 0)

  def writeback(dst_hbm):
    pltpu.sync_copy(acc.at[pl.ds(rbase, ROWS_PER_TILE)],
                    dst_hbm.at[pl.ds(rbase, ROWS_PER_TILE)])

  def combine(x_hbm, l1_hbm, l2_hbm, fin_hbm):
    def chunk(j, _):
      rs = rbase + j * RB
      pltpu.sync_copy(x_hbm.at[pl.ds(rs, RB)], za_v)
      pltpu.sync_copy(l1_hbm.at[pl.ds(rs, RB)], zb_v)
      pltpu.sync_copy(l2_hbm.at[pl.ds(rs, RB)], zc_v)
      pltpu.sync_copy(acc.at[pl.ds(rs, RB)], zd_v)
      def body(r, _):
        for h in (0, 16):
          s = (za_v[r, pl.ds(h, 16)] + zb_v[r, pl.ds(h, 16)]
               + zc_v[r, pl.ds(h, 16)] + zd_v[r, pl.ds(h, 16)])
          za_v[r, pl.ds(h, 16)] = s * 0.25
        return 0
      lax.fori_loop(0, RB, body, 0)
      pltpu.sync_copy(za_v, fin_hbm.at[pl.ds(rs, RB)])
      return 0
    lax.fori_loop(0, N_RBLK, chunk, 0)

  def propagate(x_hbm, l1_hbm, l2_hbm, fin_hbm):
    fill_zeros(zz_v)
    zero_acc_slice()
    plsc.subcore_barrier()
    edge_pass(x_hbm)
    plsc.subcore_barrier()
    writeback(l1_hbm)
    zero_acc_slice()
    plsc.subcore_barrier()
    edge_pass(l1_hbm)
    plsc.subcore_barrier()
    writeback(l2_hbm)
    zero_acc_slice()
    plsc.subcore_barrier()
    edge_pass(l2_hbm)
    plsc.subcore_barrier()
    combine(x_hbm, l1_hbm, l2_hbm, fin_hbm)

  @pl.when(cid == 0)
  def _():
    propagate(x_lo, l1_lo, l2_lo, fin_lo)

  @pl.when(cid == 1)
  def _():
    propagate(x_hi, l1_hi, l2_hi, fin_hi)


def kernel(user_emb, item_emb, graph_indices, graph_values):
  all_emb = jnp.concatenate([user_emb, item_emb], axis=0)
  x_lo = all_emb[:, :DH]
  x_hi = all_emb[:, DH:]
  row = graph_indices[0].reshape(NNZ // EB, EB)
  col = graph_indices[1].reshape(NNZ // EB, EB)
  val = graph_values.reshape(NNZ // EB, EB)
  fin_lo, fin_hi, _, _, _, _ = _lightgcn_sc(col, row, val, x_lo, x_hi)
  light_out = jnp.concatenate([fin_lo, fin_hi], axis=1)
  n_users = user_emb.shape[0]
  return light_out[:n_users], light_out[n_users:]


# speedup vs baseline: 12.5055x; 1.3605x over previous
import functools

import jax
import jax.numpy as jnp
from jax import lax
from jax.experimental import pallas as pl
from jax.experimental.pallas import tpu as pltpu
from jax.experimental.pallas import tpu_sc as plsc

N_TOTAL = 50000
D = 64
DH = 32
NNZ = 800000
NS = 16
EB = 80
BLKS_PER_TILE = NNZ // NS // EB
NBUF = 5
NSC = 4
CH = 25
N_CHUNK = BLKS_PER_TILE // CH
OUTER = CH // NBUF
ROWS_PER_TILE = N_TOTAL // NS
N_RBLK = ROWS_PER_TILE // EB
R_TAIL = ROWS_PER_TILE - N_RBLK * EB

_mesh = plsc.VectorSubcoreMesh(core_axis_name="c", subcore_axis_name="s")

_f32 = jnp.float32
_out = jax.ShapeDtypeStruct((N_TOTAL, DH), _f32)


@functools.partial(
    pl.kernel,
    out_type=(_out, _out, _out, _out, _out, _out),
    mesh=_mesh,
    compiler_params=pltpu.CompilerParams(use_tc_tiling_on_sc=False),
    scratch_types=[
        pltpu.VMEM_SHARED((N_TOTAL, DH), _f32),
        pltpu.VMEM((CH, EB), jnp.int32),
        pltpu.VMEM((CH, EB), jnp.int32),
        pltpu.VMEM((CH, EB), _f32),
        [pltpu.VMEM((EB, DH), _f32)] * NBUF,
        [pltpu.SemaphoreType.DMA] * NBUF,
        [pltpu.VMEM((EB, DH), _f32)] * NSC,
        [pltpu.SemaphoreType.DMA] * NSC,
    ],
)
def _lightgcn_sc(col_hbm, row_hbm, val_hbm, x_lo, x_hi,
                 fin_lo, fin_hi, l1_lo, l1_hi, l2_lo, l2_hi,
                 acc, col_v, row_v, val_v, ga, gs, sc, ss):
  cid = lax.axis_index("c")
  tid = lax.axis_index("s")
  rbase = tid * ROWS_PER_TILE
  zeros16 = jnp.zeros((16,), _f32)

  def fill_zeros(buf):
    def body(r, _):
      buf[r, pl.ds(0, 16)] = zeros16
      buf[r, pl.ds(16, 16)] = zeros16
      return 0
    lax.fori_loop(0, EB, body, 0)

  def zero_acc_slice():
    def body(j, _):
      pltpu.sync_copy(ga[0], acc.at[pl.ds(rbase + j * EB, EB)])
      return 0
    lax.fori_loop(0, N_RBLK, body, 0)
    pltpu.sync_copy(ga[0].at[pl.ds(0, R_TAIL)],
                    acc.at[pl.ds(rbase + N_RBLK * EB, R_TAIL)])

  def scale(src, dst, vrow):
    def grp(g, _):
      vv = val_v[vrow, pl.ds(g * 16, 16)]
      e0 = g * 16
      for j in range(16):
        v = vv[j]
        dst[e0 + j, pl.ds(0, 16)] = src[e0 + j, pl.ds(0, 16)] * v
        dst[e0 + j, pl.ds(16, 16)] = src[e0 + j, pl.ds(16, 16)] * v
      return 0
    lax.fori_loop(0, EB // 16, grp, 0)

  def edge_pass(src_hbm):
    def chunk(ci, _):
      crow = tid * BLKS_PER_TILE + ci * CH
      pltpu.sync_copy(col_hbm.at[pl.ds(crow, CH)], col_v)
      pltpu.sync_copy(row_hbm.at[pl.ds(crow, CH)], row_v)
      pltpu.sync_copy(val_hbm.at[pl.ds(crow, CH)], val_v)
      for k in range(NBUF):
        pltpu.async_copy(src_hbm.at[col_v.at[k]], ga[k], gs[k])
      def outer(oi, _):
        for k in range(NBUF):
          s = k % NSC
          j = oi * NBUF + k
          pltpu.make_async_copy(src_hbm.at[col_v.at[j]], ga[k], gs[k]).wait()
          if k < NSC:
            @pl.when(jnp.logical_or(ci > 0, oi > 0))
            def _():
              pltpu.make_async_copy(sc[s], acc.at[row_v.at[j]], ss[s]).wait()
          else:
            pltpu.make_async_copy(sc[s], acc.at[row_v.at[j]], ss[s]).wait()
          scale(ga[k], sc[s], j)
          pltpu.async_copy(sc[s], acc.at[row_v.at[j]], ss[s], add=True)
          @pl.when(oi < OUTER - 1)
          def _():
            pltpu.async_copy(src_hbm.at[col_v.at[j + NBUF]], ga[k], gs[k])
        return 0
      lax.fori_loop(0, OUTER, outer, 0)
      return 0
    lax.fori_loop(0, N_CHUNK, chunk, 0)
    for s in range(NSC):
      pltpu.make_async_copy(sc[s], acc.at[row_v.at[0]], ss[s]).wait()

  def writeback(dst_hbm):
    pltpu.sync_copy(acc.at[pl.ds(rbase, ROWS_PER_TILE)],
                    dst_hbm.at[pl.ds(rbase, ROWS_PER_TILE)])

  def combine(x_hbm, l1_hbm, l2_hbm, fin_hbm):
    def do_rows(rs, nrows):
      pltpu.sync_copy(x_hbm.at[pl.ds(rs, nrows)], sc[0].at[pl.ds(0, nrows)])
      pltpu.sync_copy(l1_hbm.at[pl.ds(rs, nrows)], sc[1].at[pl.ds(0, nrows)])
      pltpu.sync_copy(l2_hbm.at[pl.ds(rs, nrows)], sc[2].at[pl.ds(0, nrows)])
      pltpu.sync_copy(acc.at[pl.ds(rs, nrows)], sc[3].at[pl.ds(0, nrows)])
      def body(r, _):
        for h in (0, 16):
          t = (sc[0][r, pl.ds(h, 16)] + sc[1][r, pl.ds(h, 16)]
               + sc[2][r, pl.ds(h, 16)] + sc[3][r, pl.ds(h, 16)])
          sc[0][r, pl.ds(h, 16)] = t * 0.25
        return 0
      lax.fori_loop(0, nrows, body, 0)
      pltpu.sync_copy(sc[0].at[pl.ds(0, nrows)], fin_hbm.at[pl.ds(rs, nrows)])
    def chunkc(j, _):
      do_rows(rbase + j * EB, EB)
      return 0
    lax.fori_loop(0, N_RBLK, chunkc, 0)
    do_rows(rbase + N_RBLK * EB, R_TAIL)

  def propagate(x_hbm, l1_hbm, l2_hbm, fin_hbm):
    fill_zeros(ga[0])
    zero_acc_slice()
    plsc.subcore_barrier()
    edge_pass(x_hbm)
    plsc.subcore_barrier()
    writeback(l1_hbm)
    fill_zeros(ga[0])
    zero_acc_slice()
    plsc.subcore_barrier()
    edge_pass(l1_hbm)
    plsc.subcore_barrier()
    writeback(l2_hbm)
    fill_zeros(ga[0])
    zero_acc_slice()
    plsc.subcore_barrier()
    edge_pass(l2_hbm)
    plsc.subcore_barrier()
    combine(x_hbm, l1_hbm, l2_hbm, fin_hbm)

  @pl.when(cid == 0)
  def _():
    propagate(x_lo, l1_lo, l2_lo, fin_lo)

  @pl.when(cid == 1)
  def _():
    propagate(x_hi, l1_hi, l2_hi, fin_hi)


def kernel(user_emb, item_emb, graph_indices, graph_values):
  all_emb = jnp.concatenate([user_emb, item_emb], axis=0)
  x_lo = all_emb[:, :DH]
  x_hi = all_emb[:, DH:]
  row = graph_indices[0].reshape(NNZ // EB, EB)
  col = graph_indices[1].reshape(NNZ // EB, EB)
  val = graph_values.reshape(NNZ // EB, EB)
  fin_lo, fin_hi, _, _, _, _ = _lightgcn_sc(col, row, val, x_lo, x_hi)
  light_out = jnp.concatenate([fin_lo, fin_hi], axis=1)
  n_users = user_emb.shape[0]
  return light_out[:n_users], light_out[n_users:]
